# baseline (speedup 1.0000x reference)
import functools

import jax
import jax.numpy as jnp
from jax import lax
from jax.experimental import pallas as pl
from jax.experimental.pallas import tpu as pltpu
from jax.experimental.pallas import tpu_sc as plsc

B, T = 1024, 200
NRG, NSG, NPC, EMB = 32, 32, 64, 128


TB = 8
GRID = T // TB


def _tc_body(r_ref, s_ref, p_ref, w_ref, emb_ref):
    r = r_ref[...][:, None, :]
    s = s_ref[...][:, None, :]
    p = p_ref[...][:, None, :]
    i128 = lax.broadcasted_iota(jnp.int32, (TB, EMB, B), 1)
    tg = ((i128 == r) | (i128 == s + NRG) | (i128 == p + NRG + NSG)
          ).astype(jnp.float32)
    w = w_ref[...]
    for tt in range(TB):
        emb_ref[tt] = jnp.dot(w, tg[tt], preferred_element_type=jnp.float32)


def _tc_emb(rT, sT, pT, W):
    idx_spec = pl.BlockSpec((TB, B), lambda i: (i, 0))
    return pl.pallas_call(
        _tc_body,
        grid=(GRID,),
        in_specs=[idx_spec, idx_spec, idx_spec,
                  pl.BlockSpec((EMB, EMB), lambda i: (0, 0))],
        out_specs=pl.BlockSpec((TB, EMB, B), lambda i: (i, 0, 0)),
        out_shape=jax.ShapeDtypeStruct((T, EMB, B), jnp.float32),
    )(rT, sT, pT, W)


NC, NS = 2, 16
NW = NC * NS
TW = T // 8
KW = 32
NJ = B // 16

_sc_mesh = plsc.VectorSubcoreMesh(core_axis_name="c", subcore_axis_name="s")


@functools.partial(
    pl.kernel,
    out_type=(jax.ShapeDtypeStruct((T, NRG, B), jnp.float32),
              jax.ShapeDtypeStruct((T, NSG, B), jnp.float32),
              jax.ShapeDtypeStruct((T, NPC, B), jnp.float32)),
    mesh=_sc_mesh,
    scratch_types=[pltpu.VMEM((B,), jnp.int32),
                   pltpu.VMEM((KW, B), jnp.float32),
                   pltpu.VMEM((KW, B), jnp.float32),
                   pltpu.SemaphoreType.DMA,
                   pltpu.SemaphoreType.DMA],
)
def _sc_onehots(rT_hbm, sT_hbm, pT_hbm, r_out, s_out, p_out,
                ivq, slabA, slabB, semA, semB):
    wid = lax.axis_index("s") * NC + lax.axis_index("c")
    which = wid >> 3
    t0 = (wid & 7) * TW

    one = jnp.full((16,), 1.0, jnp.float32)
    zero = jnp.zeros((16,), jnp.float32)

    def run(src_hbm, dst_at, kbase):

        def build(slab, t):
            pltpu.sync_copy(src_hbm.at[t], ivq)

            def inj(j, c):
                v = ivq[pl.ds(j * 16, 16)]
                for kk in range(KW):
                    slab[kk, pl.ds(j * 16, 16)] = jnp.where(
                        v == kbase + kk, one, zero)
                return c

            lax.fori_loop(0, NJ, inj, 0)

        def step(i, c):
            t = t0 + i

            @pl.when((i & 1) == 0)
            def _():
                @pl.when(i >= 2)
                def _():
                    pltpu.make_async_copy(slabA, dst_at(t0), semA).wait()
                build(slabA, t)
                pltpu.async_copy(slabA, dst_at(t), semA)

            @pl.when((i & 1) == 1)
            def _():
                @pl.when(i >= 2)
                def _():
                    pltpu.make_async_copy(slabB, dst_at(t0), semB).wait()
                build(slabB, t)
                pltpu.async_copy(slabB, dst_at(t), semB)

            return c

        lax.fori_loop(0, TW, step, 0)
        pltpu.make_async_copy(slabA, dst_at(t0), semA).wait()
        pltpu.make_async_copy(slabB, dst_at(t0), semB).wait()

    lax.switch(which, [
        lambda: run(rT_hbm, lambda t: r_out.at[t], 0),
        lambda: run(sT_hbm, lambda t: s_out.at[t], 0),
        lambda: run(pT_hbm, lambda t: p_out.at[t, pl.ds(0, KW)], 0),
        lambda: run(pT_hbm, lambda t: p_out.at[t, pl.ds(KW, KW)], KW),
    ])


def kernel(rgap, sgap, pcount, W):
    rT = rgap.T
    sT = sgap.T
    pT = pcount.T
    r_oh, s_oh, p_oh = _sc_onehots(rT, sT, pT)
    emb = _tc_emb(rT, sT, pT, W)
    return (r_oh.transpose(2, 0, 1), s_oh.transpose(2, 0, 1),
            p_oh.transpose(2, 0, 1), emb.transpose(2, 0, 1))


# speedup vs baseline: 2.0545x; 2.0545x over previous
import functools

import jax
import jax.numpy as jnp
from jax import lax
from jax.experimental import pallas as pl
from jax.experimental.pallas import tpu as pltpu
from jax.experimental.pallas import tpu_sc as plsc

B, T = 1024, 200
NRG, NSG, NPC, EMB = 32, 32, 64, 128


TB = 8
GRID = T // TB


def _tc_body(r_ref, s_ref, p_ref, w_ref, emb_ref):
    r = r_ref[...][:, None, :]
    s = s_ref[...][:, None, :]
    p = p_ref[...][:, None, :]
    i128 = lax.broadcasted_iota(jnp.int32, (TB, EMB, B), 1)
    tg = ((i128 == r) | (i128 == s + NRG) | (i128 == p + NRG + NSG)
          ).astype(jnp.float32)
    w = w_ref[...]
    for tt in range(TB):
        emb_ref[:, tt, :] = lax.dot_general(
            tg[tt], w, (((0,), (1,)), ((), ())),
            preferred_element_type=jnp.float32)


def _tc_emb(rT, sT, pT, W):
    idx_spec = pl.BlockSpec((TB, B), lambda i: (i, 0))
    return pl.pallas_call(
        _tc_body,
        grid=(GRID,),
        in_specs=[idx_spec, idx_spec, idx_spec,
                  pl.BlockSpec((EMB, EMB), lambda i: (0, 0))],
        out_specs=pl.BlockSpec((B, TB, EMB), lambda i: (0, i, 0)),
        out_shape=jax.ShapeDtypeStruct((B, T, EMB), jnp.float32),
    )(rT, sT, pT, W)


NC, NS = 2, 16
NW = NC * NS
TW = T // 8
KW = 32
NJ = B // 16

_sc_mesh = plsc.VectorSubcoreMesh(core_axis_name="c", subcore_axis_name="s")


@functools.partial(
    pl.kernel,
    out_type=(jax.ShapeDtypeStruct((T, NRG, B), jnp.float32),
              jax.ShapeDtypeStruct((T, NSG, B), jnp.float32),
              jax.ShapeDtypeStruct((T, NPC, B), jnp.float32)),
    mesh=_sc_mesh,
    scratch_types=[pltpu.VMEM((B,), jnp.int32),
                   pltpu.VMEM((KW, B), jnp.float32),
                   pltpu.VMEM((KW, B), jnp.float32),
                   pltpu.SemaphoreType.DMA,
                   pltpu.SemaphoreType.DMA],
)
def _sc_onehots(rT_hbm, sT_hbm, pT_hbm, r_out, s_out, p_out,
                ivq, slabA, slabB, semA, semB):
    wid = lax.axis_index("s") * NC + lax.axis_index("c")
    which = wid >> 3
    t0 = (wid & 7) * TW

    one = jnp.full((16,), 1.0, jnp.float32)
    zero = jnp.zeros((16,), jnp.float32)

    def run(src_hbm, dst_at, kbase):

        def build(slab, t):
            pltpu.sync_copy(src_hbm.at[t], ivq)

            def inj(j, c):
                v = ivq[pl.ds(j * 16, 16)]
                for kk in range(KW):
                    slab[kk, pl.ds(j * 16, 16)] = jnp.where(
                        v == kbase + kk, one, zero)
                return c

            lax.fori_loop(0, NJ, inj, 0)

        def step(i, c):
            t = t0 + i

            @pl.when((i & 1) == 0)
            def _():
                @pl.when(i >= 2)
                def _():
                    pltpu.make_async_copy(slabA, dst_at(t0), semA).wait()
                build(slabA, t)
                pltpu.async_copy(slabA, dst_at(t), semA)

            @pl.when((i & 1) == 1)
            def _():
                @pl.when(i >= 2)
                def _():
                    pltpu.make_async_copy(slabB, dst_at(t0), semB).wait()
                build(slabB, t)
                pltpu.async_copy(slabB, dst_at(t), semB)

            return c

        lax.fori_loop(0, TW, step, 0)
        pltpu.make_async_copy(slabA, dst_at(t0), semA).wait()
        pltpu.make_async_copy(slabB, dst_at(t0), semB).wait()

    lax.switch(which, [
        lambda: run(rT_hbm, lambda t: r_out.at[t], 0),
        lambda: run(sT_hbm, lambda t: s_out.at[t], 0),
        lambda: run(pT_hbm, lambda t: p_out.at[t, pl.ds(0, KW)], 0),
        lambda: run(pT_hbm, lambda t: p_out.at[t, pl.ds(KW, KW)], KW),
    ])


def kernel(rgap, sgap, pcount, W):
    rT = rgap.T
    sT = sgap.T
    pT = pcount.T
    r_oh, s_oh, p_oh = _sc_onehots(rT, sT, pT)
    emb = _tc_emb(rT, sT, pT, W)
    return (r_oh.transpose(2, 0, 1), s_oh.transpose(2, 0, 1),
            p_oh.transpose(2, 0, 1), emb)
